# baseline (speedup 1.0000x reference)
import functools

import jax
import jax.numpy as jnp
from jax import lax
from jax.experimental import pallas as pl
from jax.experimental.pallas import tpu as pltpu
from jax.experimental.pallas import tpu_sc as plsc

B = 16384
D = 128
NC = 2
NS = 16
L = 16
NW = NC * NS
BPW = B // NW
PPAD = 16
VPR = D // L
NCHUNK = 4
CH = BPW // NCHUNK
GPC = CH // L
NGRP = BPW // L


def _sc_fused(cs, params_pad, xs):
    mesh = plsc.VectorSubcoreMesh(core_axis_name="c", subcore_axis_name="s")

    @functools.partial(
        pl.kernel,
        out_type=jax.ShapeDtypeStruct((B, D), jnp.float32),
        mesh=mesh,
        scratch_types=[
            pltpu.VMEM((BPW,), jnp.int32),
            pltpu.VMEM((PPAD,), jnp.float32),
            pltpu.VMEM((BPW,), jnp.float32),
            pltpu.VMEM((BPW, D), jnp.float32),
            [pltpu.SemaphoreType.DMA] * NCHUNK,
            [pltpu.SemaphoreType.DMA] * NCHUNK,
        ],
    )
    def k(cs_hbm, p_hbm, xs_hbm, out_hbm, cs_v, p_v, g_v, x_v, sin, sout):
        wid = lax.axis_index("s") * NC + lax.axis_index("c")
        base = wid * BPW

        def in_copy(t):
            return pltpu.make_async_copy(
                xs_hbm.at[pl.ds(base + t * CH, CH)],
                x_v.at[pl.ds(t * CH, CH)],
                sin[t],
            )

        def out_copy(t):
            return pltpu.make_async_copy(
                x_v.at[pl.ds(t * CH, CH)],
                out_hbm.at[pl.ds(base + t * CH, CH)],
                sout[t],
            )

        for t in range(NCHUNK):
            in_copy(t).start()

        pltpu.sync_copy(p_hbm, p_v)
        pltpu.sync_copy(cs_hbm.at[pl.ds(base, BPW)], cs_v)
        p_vec = p_v[...]

        def gather_body(i, carry):
            idx = cs_v[pl.ds(i * L, L)]
            g_v[pl.ds(i * L, L)] = jnp.take_along_axis(
                p_vec, idx, axis=0, mode="promise_in_bounds"
            )
            return carry

        lax.fori_loop(0, NGRP, gather_body, 0)

        def grp_body(g, carry):
            for t in range(NCHUNK):
                @pl.when(g == t * GPC)
                def _():
                    in_copy(t).wait()

            r0 = g * L
            g16 = g_v[pl.ds(r0, L)]
            for j in range(L):
                s = jnp.take_along_axis(
                    g16, jnp.full((L,), j, jnp.int32), axis=0,
                    mode="promise_in_bounds",
                )
                for c in range(VPR):
                    x_v[r0 + j, pl.ds(c * L, L)] = (
                        x_v[r0 + j, pl.ds(c * L, L)] * s
                    )

            for t in range(NCHUNK):
                @pl.when(g == t * GPC + (GPC - 1))
                def _():
                    out_copy(t).start()
            return carry

        lax.fori_loop(0, NGRP, grp_body, 0)

        for t in range(NCHUNK):
            out_copy(t).wait()

    return k(cs, params_pad, xs)


def kernel(cs, xs, params):
    flat = params.reshape(-1)
    p_pad = jnp.zeros((PPAD,), jnp.float32).at[: flat.shape[0]].set(flat)
    return _sc_fused(cs, p_pad, xs)
